# Initial kernel scaffold; baseline (speedup 1.0000x reference)
import jax
import jax.numpy as jnp
from jax.experimental import pallas as pl


def kernel(queries, keys):
    raise NotImplementedError("write your pallas kernel here")


# speedup vs baseline: 4.8632x; 4.8632x over previous
import functools

import jax
import jax.numpy as jnp
from jax.experimental import pallas as pl
from jax.experimental.pallas import tpu as pltpu

Q = 1024
D = 16
K = 100000
KB = 2048
NBLK = (K + KB - 1) // KB

_BIG = 1e30


def _knn_kernel(q_ref, k_ref, scores_ref, idx_ref, img_ref, min_s, idx_s):
    j = pl.program_id(0)
    nblk = pl.num_programs(0)

    @pl.when(j == 0)
    def _init():
        min_s[...] = jnp.full((1, Q), jnp.float32(_BIG))
        idx_s[...] = jnp.zeros((1, Q), dtype=jnp.int32)

    qt = q_ref[...]
    k = k_ref[...]

    q_sq = jnp.sum(qt * qt, axis=0, keepdims=True)
    k_sq = jnp.sum(k * k, axis=1, keepdims=True)

    row = jax.lax.broadcasted_iota(jnp.int32, (KB, 1), 0) + j * KB
    k_sq = jnp.where(row < K, k_sq, jnp.float32(_BIG))

    qk = jax.lax.dot_general(k.astype(jnp.bfloat16), qt.astype(jnp.bfloat16),
                             (((1,), (0,)), ((), ())),
                             preferred_element_type=jnp.float32)
    d = (k_sq + q_sq) - 2.0 * qk

    bmin = jnp.min(d, axis=0, keepdims=True)
    col = jax.lax.broadcasted_iota(jnp.int32, (KB, Q), 0)
    bidx = jnp.min(jnp.where(d == bmin, col, jnp.int32(2**30)),
                   axis=0, keepdims=True) + j * KB

    run_min = min_s[...]
    upd = bmin < run_min
    min_s[...] = jnp.where(upd, bmin, run_min)
    idx_s[...] = jnp.where(upd, bidx, idx_s[...])

    @pl.when(j == nblk - 1)
    def _fin():
        final = min_s[...]
        scores_ref[...] = final
        idx_ref[...] = idx_s[...]
        img_ref[...] = jnp.max(final).reshape(1, 1)


@functools.partial(jax.jit, static_argnames=())
def _knn(queries, keys):
    return pl.pallas_call(
        _knn_kernel,
        grid=(NBLK,),
        in_specs=[
            pl.BlockSpec((D, Q), lambda j: (0, 0)),
            pl.BlockSpec((KB, D), lambda j: (j, 0)),
        ],
        out_specs=[
            pl.BlockSpec((1, Q), lambda j: (0, 0)),
            pl.BlockSpec((1, Q), lambda j: (0, 0)),
            pl.BlockSpec((1, 1), lambda j: (0, 0)),
        ],
        out_shape=[
            jax.ShapeDtypeStruct((1, Q), jnp.float32),
            jax.ShapeDtypeStruct((1, Q), jnp.int32),
            jax.ShapeDtypeStruct((1, 1), jnp.float32),
        ],
        scratch_shapes=[
            pltpu.VMEM((1, Q), jnp.float32),
            pltpu.VMEM((1, Q), jnp.int32),
        ],
        compiler_params=pltpu.CompilerParams(
            dimension_semantics=("arbitrary",),
        ),
    )(queries, keys)


def kernel(queries, keys):
    scores, idx, img = _knn(queries.T, keys)
    patch_scores = scores.reshape(Q)
    nn_idx = idx.reshape(Q, 1)
    image_score = img.reshape(())
    return patch_scores, image_score, nn_idx
